# Initial kernel scaffold; baseline (speedup 1.0000x reference)
import jax
import jax.numpy as jnp
from jax.experimental import pallas as pl


def kernel(X, Candidate, neigh_dist, neigh_ind, data_m_train, data_m_batch, test, W, a):
    raise NotImplementedError("write your pallas kernel here")


# speedup vs baseline: 3.9564x; 3.9564x over previous
import functools

import jax
import jax.numpy as jnp
from jax import lax
from jax.experimental import pallas as pl
from jax.experimental.pallas import tpu as pltpu
from jax.experimental.pallas import tpu_sc as plsc

NO, DIM, K0, N_TRAIN, OUT_F, K_SEL = 8192, 64, 51, 100000, 16, 50

_NC, _NS = 2, 16
_NW = _NC * _NS
_N_IDX = NO * K0
_CHUNK = _N_IDX // _NW
_SUB = 1632
_NSUB = _CHUNK // _SUB

_BR = 128
_NCOL = 8


def _sc_gather_body(tab_hbm, idx_hbm, out_hbm, idx_v, buf_v, sem):
    wid = lax.axis_index("s") * _NC + lax.axis_index("c")
    base = wid * _CHUNK
    pltpu.sync_copy(idx_hbm.at[pl.ds(base, _CHUNK)], idx_v)
    for k in range(_NSUB):
        pltpu.async_copy(tab_hbm.at[idx_v.at[pl.ds(k * _SUB, _SUB)]], buf_v,
                         sem).wait()
        pltpu.sync_copy(buf_v, out_hbm.at[pl.ds(base + k * _SUB, _SUB)])


@functools.cache
def _make_sc_gather():
    return functools.partial(
        pl.kernel,
        out_type=jax.ShapeDtypeStruct((_N_IDX, DIM), jnp.float32),
        mesh=plsc.VectorSubcoreMesh(core_axis_name="c", subcore_axis_name="s"),
        scratch_types=[
            pltpu.VMEM((_CHUNK,), jnp.int32),
            pltpu.VMEM((_SUB, DIM), jnp.float32),
            pltpu.SemaphoreType.DMA,
        ],
        compiler_params=pltpu.CompilerParams(use_tc_tiling_on_sc=False),
    )(_sc_gather_body)


def _sel_body(x_ref, dmb_ref, w_ref, a_ref, c_ref, dmc_ref, nd_ref, ni_ref,
              cs_ref, nds_ref, nis_ref, acc_ref):
    C = c_ref[...]
    dmc = dmc_ref[...]
    br = C.shape[0]
    wb = w_ref[...].astype(jnp.bfloat16)
    ab = lax.broadcast_in_dim(a_ref[...], (2 * OUT_F, _NCOL),
                              (0, 1)).astype(jnp.bfloat16)
    xc = jnp.concatenate([x_ref[...], dmb_ref[...]],
                         axis=1).astype(jnp.bfloat16)
    wx = lax.dot_general(xc, wb, (((1,), (0,)), ((), ())),
                         preferred_element_type=jnp.float32)
    cm = jnp.concatenate([C, dmc], axis=2).astype(jnp.bfloat16)
    cm2 = cm.reshape(br * K0, 2 * DIM)
    wc = lax.dot_general(cm2, wb, (((1,), (0,)), ((), ())),
                         preferred_element_type=jnp.float32)
    wxr = lax.broadcast_in_dim(wx, (br, K0, OUT_F),
                               (0, 2)).reshape(br * K0, OUT_F)
    inp = jnp.concatenate([wxr, wc], axis=1).astype(jnp.bfloat16)
    e = lax.dot_general(inp, ab, (((1,), (0,)), ((), ())),
                        preferred_element_type=jnp.float32)
    s3 = e.reshape(br, K0, _NCOL)
    mn = jnp.min(s3, axis=1, keepdims=True)
    jj = lax.broadcasted_iota(jnp.int32, s3.shape, 1)
    dropc = jnp.max(jnp.where(s3 == mn, jj, -1), axis=1)
    drop2 = dropc[:, :1]

    j50 = lax.broadcasted_iota(jnp.int32, (br, K_SEL), 1)
    keep_lo = j50 < lax.broadcast_in_dim(drop2, (br, K_SEL), (0, 1))
    j3 = lax.broadcasted_iota(jnp.int32, (br, K_SEL, DIM), 1)
    keep3 = j3 < lax.broadcast_in_dim(drop2, (br, K_SEL, DIM), (0, 1))
    cs_ref[...] = jnp.where(keep3, C[:, :K_SEL, :], C[:, 1:, :])
    nd = nd_ref[...]
    nds_ref[...] = jnp.where(keep_lo, nd[:, :K_SEL], nd[:, 1:])
    ni = ni_ref[...]
    nis_ref[...] = jnp.where(keep_lo, ni[:, :K_SEL], ni[:, 1:])

    jd = lax.broadcasted_iota(jnp.int32, (br, K0, DIM), 1)
    dm = lax.broadcast_in_dim(drop2, (br, K0, DIM), (0, 1))
    rdrop = jnp.sum(jnp.where(jd == dm, dmc, 0.0))
    rall = jnp.sum(dmc)

    @pl.when(pl.program_id(0) == 0)
    def _():
        acc_ref[...] = jnp.zeros((1, 2), jnp.float32)

    lane2 = lax.broadcasted_iota(jnp.int32, (1, 2), 1)
    acc_ref[...] += jnp.where(lane2 == 0, rall - rdrop, rdrop)


def _select(X, dmb, W, a, Candidate, dmc, neigh_dist, neigh_ind):
    grid = NO // _BR
    return pl.pallas_call(
        _sel_body,
        grid=(grid,),
        in_specs=[
            pl.BlockSpec((_BR, DIM), lambda i: (i, 0)),
            pl.BlockSpec((_BR, DIM), lambda i: (i, 0)),
            pl.BlockSpec((2 * DIM, OUT_F), lambda i: (0, 0)),
            pl.BlockSpec((2 * OUT_F, 1), lambda i: (0, 0)),
            pl.BlockSpec((_BR, K0, DIM), lambda i: (i, 0, 0)),
            pl.BlockSpec((_BR, K0, DIM), lambda i: (i, 0, 0)),
            pl.BlockSpec((_BR, K0), lambda i: (i, 0)),
            pl.BlockSpec((_BR, K0), lambda i: (i, 0)),
        ],
        out_specs=[
            pl.BlockSpec((_BR, K_SEL, DIM), lambda i: (i, 0, 0)),
            pl.BlockSpec((_BR, K_SEL), lambda i: (i, 0)),
            pl.BlockSpec((_BR, K_SEL), lambda i: (i, 0)),
            pl.BlockSpec((1, 2), lambda i: (0, 0)),
        ],
        out_shape=[
            jax.ShapeDtypeStruct((NO, K_SEL, DIM), jnp.float32),
            jax.ShapeDtypeStruct((NO, K_SEL), jnp.float32),
            jax.ShapeDtypeStruct((NO, K_SEL), jnp.int32),
            jax.ShapeDtypeStruct((1, 2), jnp.float32),
        ],
    )(X, dmb, W, a, Candidate, dmc, neigh_dist, neigh_ind)


def kernel(X, Candidate, neigh_dist, neigh_ind, data_m_train, data_m_batch,
           test, W, a):
    del test
    idx_flat = neigh_ind.reshape(_N_IDX).astype(jnp.int32)
    dmc = _make_sc_gather()(data_m_train, idx_flat)
    dmc = dmc.reshape(NO, K0, DIM)
    cs, nds, nis, acc = _select(X, data_m_batch, W, a, Candidate, dmc,
                                neigh_dist, neigh_ind)
    a_out = acc[0, 0] / jnp.float32(NO * K_SEL)
    b_out = acc[0, 1] / jnp.float32(NO)
    return cs, nds, nis, a_out, b_out


# speedup vs baseline: 4.0471x; 1.0229x over previous
import functools

import jax
import jax.numpy as jnp
from jax import lax
from jax.experimental import pallas as pl
from jax.experimental.pallas import tpu as pltpu
from jax.experimental.pallas import tpu_sc as plsc

NO, DIM, K0, N_TRAIN, OUT_F, K_SEL = 8192, 64, 51, 100000, 16, 50

_NC, _NS = 2, 16
_NW = _NC * _NS
_N_IDX = NO * K0
_CHUNK = _N_IDX // _NW
_SUB = 1632
_NSUB = _CHUNK // _SUB

_BR = 128
_NCOL = 8


def _sc_gather_body(tab_hbm, idx_hbm, out_hbm, idx_v, buf_v, sem):
    wid = lax.axis_index("s") * _NC + lax.axis_index("c")
    base = wid * _CHUNK
    pltpu.sync_copy(idx_hbm.at[pl.ds(base, _CHUNK)], idx_v)
    for k in range(_NSUB):
        pltpu.async_copy(tab_hbm.at[idx_v.at[pl.ds(k * _SUB, _SUB)]], buf_v,
                         sem).wait()
        pltpu.sync_copy(buf_v, out_hbm.at[pl.ds(base + k * _SUB, _SUB)])


@functools.cache
def _make_sc_gather():
    return functools.partial(
        pl.kernel,
        out_type=jax.ShapeDtypeStruct((_N_IDX, DIM), jnp.float32),
        mesh=plsc.VectorSubcoreMesh(core_axis_name="c", subcore_axis_name="s"),
        scratch_types=[
            pltpu.VMEM((_CHUNK,), jnp.int32),
            pltpu.VMEM((_SUB, DIM), jnp.float32),
            pltpu.SemaphoreType.DMA,
        ],
        compiler_params=pltpu.CompilerParams(use_tc_tiling_on_sc=False),
    )(_sc_gather_body)


def _sel_body(x_ref, dmb_ref, w_ref, a_ref, c_ref, dmcf_ref, nd_ref, ni_ref,
              cs_ref, nds_ref, nis_ref, nidrop_ref, acc_ref):
    C = c_ref[...]
    dmcf = dmcf_ref[...]
    br = C.shape[0]
    n = br * K0
    wb = w_ref[...].astype(jnp.bfloat16)
    ab = lax.broadcast_in_dim(a_ref[...], (2 * OUT_F, _NCOL),
                              (0, 1)).astype(jnp.bfloat16)
    xc = jnp.concatenate([x_ref[...], dmb_ref[...]],
                         axis=1).astype(jnp.bfloat16)
    wx = lax.dot_general(xc, wb, (((1,), (0,)), ((), ())),
                         preferred_element_type=jnp.float32)
    cmf = jnp.concatenate([C.astype(jnp.bfloat16).reshape(n, DIM),
                           dmcf.astype(jnp.bfloat16)], axis=1)
    wc = lax.dot_general(cmf, wb, (((1,), (0,)), ((), ())),
                         preferred_element_type=jnp.float32)
    wxr = lax.broadcast_in_dim(wx, (br, K0, OUT_F),
                               (0, 2)).reshape(n, OUT_F)
    inp = jnp.concatenate([wxr, wc], axis=1).astype(jnp.bfloat16)
    e = lax.dot_general(inp, ab, (((1,), (0,)), ((), ())),
                        preferred_element_type=jnp.float32)
    s3 = e.reshape(br, K0, _NCOL)
    mn = jnp.min(s3, axis=1, keepdims=True)
    jj = lax.broadcasted_iota(jnp.int32, s3.shape, 1)
    dropc = jnp.max(jnp.where(s3 == mn, jj, -1), axis=1)
    drop2 = dropc[:, :1]

    j50 = lax.broadcasted_iota(jnp.int32, (br, K_SEL), 1)
    keep_lo = j50 < lax.broadcast_in_dim(drop2, (br, K_SEL), (0, 1))
    j3 = lax.broadcasted_iota(jnp.int32, (br, K_SEL, DIM), 1)
    keep3 = j3 < lax.broadcast_in_dim(drop2, (br, K_SEL, DIM), (0, 1))
    cs_ref[...] = jnp.where(keep3, C[:, :K_SEL, :], C[:, 1:, :])
    nd = nd_ref[...]
    nds_ref[...] = jnp.where(keep_lo, nd[:, :K_SEL], nd[:, 1:])
    ni = ni_ref[...]
    nis_ref[...] = jnp.where(keep_lo, ni[:, :K_SEL], ni[:, 1:])

    jj2 = lax.broadcasted_iota(jnp.int32, (br, K0), 1)
    dropb2 = lax.broadcast_in_dim(drop2, (br, K0), (0, 1))
    nidrop_ref[...] = jnp.sum(jnp.where(jj2 == dropb2, ni, 0), axis=1,
                              keepdims=True)
    rall = jnp.sum(dmcf)

    @pl.when(pl.program_id(0) == 0)
    def _():
        acc_ref[...] = jnp.zeros((1, 2), jnp.float32)

    lane2 = lax.broadcasted_iota(jnp.int32, (1, 2), 1)
    acc_ref[...] += jnp.where(lane2 == 0, rall, 0.0)


def _select(X, dmb, W, a, Candidate, dmc_flat, neigh_dist, neigh_ind):
    grid = NO // _BR
    return pl.pallas_call(
        _sel_body,
        grid=(grid,),
        in_specs=[
            pl.BlockSpec((_BR, DIM), lambda i: (i, 0)),
            pl.BlockSpec((_BR, DIM), lambda i: (i, 0)),
            pl.BlockSpec((2 * DIM, OUT_F), lambda i: (0, 0)),
            pl.BlockSpec((2 * OUT_F, 1), lambda i: (0, 0)),
            pl.BlockSpec((_BR, K0, DIM), lambda i: (i, 0, 0)),
            pl.BlockSpec((_BR * K0, DIM), lambda i: (i, 0)),
            pl.BlockSpec((_BR, K0), lambda i: (i, 0)),
            pl.BlockSpec((_BR, K0), lambda i: (i, 0)),
        ],
        out_specs=[
            pl.BlockSpec((_BR, K_SEL, DIM), lambda i: (i, 0, 0)),
            pl.BlockSpec((_BR, K_SEL), lambda i: (i, 0)),
            pl.BlockSpec((_BR, K_SEL), lambda i: (i, 0)),
            pl.BlockSpec((_BR, 1), lambda i: (i, 0)),
            pl.BlockSpec((1, 2), lambda i: (0, 0)),
        ],
        out_shape=[
            jax.ShapeDtypeStruct((NO, K_SEL, DIM), jnp.float32),
            jax.ShapeDtypeStruct((NO, K_SEL), jnp.float32),
            jax.ShapeDtypeStruct((NO, K_SEL), jnp.int32),
            jax.ShapeDtypeStruct((NO, 1), jnp.int32),
            jax.ShapeDtypeStruct((1, 2), jnp.float32),
        ],
    )(X, dmb, W, a, Candidate, dmc_flat, neigh_dist, neigh_ind)


_CHUNK_B = NO // _NW


def _sc_gather_b_body(tab_hbm, idx_hbm, out_hbm, idx_v, buf_v, sem):
    wid = lax.axis_index("s") * _NC + lax.axis_index("c")
    base = wid * _CHUNK_B
    pltpu.sync_copy(idx_hbm.at[pl.ds(base, _CHUNK_B)], idx_v)
    pltpu.async_copy(tab_hbm.at[idx_v], buf_v, sem).wait()
    pltpu.sync_copy(buf_v, out_hbm.at[pl.ds(base, _CHUNK_B)])


@functools.cache
def _make_sc_gather_b():
    return functools.partial(
        pl.kernel,
        out_type=jax.ShapeDtypeStruct((NO, DIM), jnp.float32),
        mesh=plsc.VectorSubcoreMesh(core_axis_name="c", subcore_axis_name="s"),
        scratch_types=[
            pltpu.VMEM((_CHUNK_B,), jnp.int32),
            pltpu.VMEM((_CHUNK_B, DIM), jnp.float32),
            pltpu.SemaphoreType.DMA,
        ],
        compiler_params=pltpu.CompilerParams(use_tc_tiling_on_sc=False),
    )(_sc_gather_b_body)


def _bsum_body(rows_ref, out_ref):
    @pl.when(pl.program_id(0) == 0)
    def _():
        out_ref[...] = jnp.zeros((1, 1), jnp.float32)

    out_ref[...] += jnp.sum(rows_ref[...])


def _bsum(rows):
    bt = 1024
    return pl.pallas_call(
        _bsum_body,
        grid=(NO // bt,),
        in_specs=[pl.BlockSpec((bt, DIM), lambda i: (i, 0))],
        out_specs=pl.BlockSpec((1, 1), lambda i: (0, 0)),
        out_shape=jax.ShapeDtypeStruct((1, 1), jnp.float32),
    )(rows)


def kernel(X, Candidate, neigh_dist, neigh_ind, data_m_train, data_m_batch,
           test, W, a):
    del test
    idx_flat = neigh_ind.reshape(_N_IDX).astype(jnp.int32)
    dmc_flat = _make_sc_gather()(data_m_train, idx_flat)
    cs, nds, nis, nidrop, acc = _select(X, data_m_batch, W, a, Candidate,
                                        dmc_flat, neigh_dist, neigh_ind)
    drows = _make_sc_gather_b()(data_m_train, nidrop.reshape(NO))
    b_sum = _bsum(drows)[0, 0]
    a_out = (acc[0, 0] - b_sum) / jnp.float32(NO * K_SEL)
    b_out = b_sum / jnp.float32(NO)
    return cs, nds, nis, a_out, b_out
